# baseline (speedup 1.0000x reference)
import functools

import jax
import jax.numpy as jnp
from jax import lax
from jax.experimental import pallas as pl
from jax.experimental.pallas import tpu as pltpu
from jax.experimental.pallas import tpu_sc as plsc

M, D = 131072, 128
TO_ADD_CONST = 1.0
NC, NS = 2, 16
NW = NC * NS
ROWS_PER_W = M // NW
SCHUNK = 512
NCH = ROWS_PER_W // SCHUNK
SNBUF = 2
CELEMS = SCHUNK * D

_mesh = plsc.VectorSubcoreMesh(core_axis_name="c", subcore_axis_name="s")


@functools.partial(
    pl.kernel,
    out_type=jax.ShapeDtypeStruct((M * D,), jnp.float32),
    mesh=_mesh,
    scratch_types=(
        [pltpu.VMEM((CELEMS,), jnp.float32)] * SNBUF
        + [pltpu.SemaphoreType.DMA] * (2 * SNBUF)
    ),
)
def _sc_copy(x_hbm, out_hbm, *rest):
    bufs = rest[:SNBUF]
    in_sems = rest[SNBUF:2 * SNBUF]
    out_sems = rest[2 * SNBUF:]

    wid = lax.axis_index("s") * NC + lax.axis_index("c")
    wbase = wid * (ROWS_PER_W * D)

    def in_cp(k):
        b = k % SNBUF
        return pltpu.make_async_copy(
            x_hbm.at[pl.ds(wbase + k * CELEMS, CELEMS)], bufs[b], in_sems[b])

    def out_cp(k):
        b = k % SNBUF
        return pltpu.make_async_copy(
            bufs[b], out_hbm.at[pl.ds(wbase + k * CELEMS, CELEMS)], out_sems[b])

    for j in range(SNBUF):
        in_cp(j).start()

    for k in range(NCH):
        in_cp(k).wait()
        out_cp(k).start()
        nk = k + SNBUF
        if nk < NCH:
            out_cp(k).wait()
            in_cp(nk).start()

    for k in range(NCH - SNBUF, NCH):
        out_cp(k).wait()


def _fix_body(idx_ref, x_ref, o_ref):
    r = idx_ref[0] % 8
    o_ref[...] = x_ref[...]
    o_ref[pl.ds(r, 1), :] = x_ref[pl.ds(r, 1), :] + TO_ADD_CONST


def _fix_row(copied, idx_arr):
    grid_spec = pltpu.PrefetchScalarGridSpec(
        num_scalar_prefetch=1,
        grid=(1,),
        in_specs=[pl.BlockSpec((8, D), lambda i, idx: (idx[0] // 8, 0))],
        out_specs=pl.BlockSpec((8, D), lambda i, idx: (idx[0] // 8, 0)),
    )
    return pl.pallas_call(
        _fix_body,
        grid_spec=grid_spec,
        out_shape=jax.ShapeDtypeStruct((M, D), jnp.float32),
        input_output_aliases={1: 0},
    )(idx_arr, copied)


def kernel(tensor, slice_index, related_index):
    idx_arr = jnp.asarray(slice_index, dtype=jnp.int32).reshape((1,))
    copied = _sc_copy(tensor.reshape(M * D)).reshape(M, D)
    out = _fix_row(copied, idx_arr)
    return (out, slice_index, related_index)
